# Initial kernel scaffold; baseline (speedup 1.0000x reference)
import jax
import jax.numpy as jnp
from jax.experimental import pallas as pl


def kernel(x, table):
    raise NotImplementedError("write your pallas kernel here")


# speedup vs baseline: 41.7871x; 41.7871x over previous
import functools
import math

import jax
import jax.numpy as jnp
from jax import lax
from jax.experimental import pallas as pl
from jax.experimental.pallas import tpu as pltpu
from jax.experimental.pallas import tpu_sc as plsc

_SCALE = 0.125
_NUM_BUCKETS = 32
_MAX_DISTANCE = 128
_NSHIFT = 16


def _build_lut_body(tabT_ref, out_ref, *, seq, lut_len, cpad):
    nb2 = _NUM_BUCKETS // 2
    max_exact = nb2 // 2
    c = lax.broadcasted_iota(jnp.int32, (_NUM_BUCKETS, cpad), 1)
    b = lax.broadcasted_iota(jnp.int32, (_NUM_BUCKETS, cpad), 0)
    n = (seq - 1) - c
    base = jnp.where(n < 0, nb2, 0).astype(jnp.int32)
    a = jnp.abs(n)
    af = jnp.maximum(a, 1).astype(jnp.float32)
    vlarge = max_exact + (
        jnp.log(af / max_exact) / math.log(_MAX_DISTANCE / max_exact) * (nb2 - max_exact)
    ).astype(jnp.int32)
    vlarge = jnp.minimum(vlarge, nb2 - 1)
    bucket = base + jnp.where(a < max_exact, a, vlarge)
    onehot = (bucket == b).astype(jnp.float32)
    vals = lax.dot_general(
        tabT_ref[...], onehot, (((1,), (0,)), ((), ())),
        preferred_element_type=jnp.float32,
    ) * _SCALE
    for s in range(_NSHIFT):
        out_ref[s] = vals[:, s:s + lut_len]


def _build_lut(tableT, seq):
    heads = tableT.shape[0]
    lut_len = 2 * seq
    cpad = ((lut_len + _NSHIFT + 127) // 128) * 128
    body = functools.partial(_build_lut_body, seq=seq, lut_len=lut_len, cpad=cpad)
    return pl.pallas_call(
        body,
        out_shape=jax.ShapeDtypeStruct((_NSHIFT, heads, lut_len), jnp.float32),
    )(tableT)


def _make_fanout(heads, seq):
    lut_len = 2 * seq
    mesh = plsc.VectorSubcoreMesh(core_axis_name="c", subcore_axis_name="s")
    rows_per_tec = seq // 2

    @functools.partial(
        pl.kernel,
        mesh=mesh,
        out_type=jax.ShapeDtypeStruct((heads * seq * seq,), jnp.float32),
        scratch_types=[
            pltpu.VMEM((_NSHIFT * lut_len,), jnp.float32),
            pltpu.SemaphoreType.DMA,
        ],
    )
    def fanout(lut_hbm, out_hbm, lut_tile, sem):
        h = lax.axis_index("s")
        half = lax.axis_index("c")
        for s in range(_NSHIFT):
            pltpu.sync_copy(
                lut_hbm.at[pl.ds(pl.multiple_of((s * heads + h) * lut_len, lut_len), lut_len)],
                lut_tile.at[pl.ds(s * lut_len, lut_len)],
            )
        i0 = half * rows_per_tec
        depth = 16

        def row_copy(k):
            i = i0 + k
            cs = (seq - 1) - i
            sh = lax.rem(cs, _NSHIFT)
            q = cs - sh
            return pltpu.make_async_copy(
                lut_tile.at[pl.ds(pl.multiple_of(sh * lut_len + q, _NSHIFT), seq)],
                out_hbm.at[pl.ds(pl.multiple_of((h * seq + i) * seq, seq), seq)],
                sem,
            )

        for k in range(depth):
            row_copy(k).start()

        def body(k, carry):
            row_copy(k - depth).wait()
            row_copy(k).start()
            return carry

        lax.fori_loop(depth, rows_per_tec, body, 0)

        def drain(k, carry):
            row_copy(k).wait()
            return carry

        lax.fori_loop(rows_per_tec - depth, rows_per_tec, drain, 0)

    return fanout


def kernel(x, table):
    seq = x.shape[-2]
    heads = table.shape[1]
    tableT = jnp.transpose(table)
    lut = _build_lut(tableT, seq)
    out = _make_fanout(heads, seq)(jnp.reshape(lut, (-1,)))
    return jnp.reshape(out, (1, heads, 1, seq, seq))
